# baseline (speedup 1.0000x reference)
import functools

import jax
import jax.numpy as jnp
from jax import lax
from jax.experimental import pallas as pl
from jax.experimental.pallas import tpu as pltpu
from jax.experimental.pallas import tpu_sc as plsc

_N = 10000
_D = 128
_E = 320000

_NC = 2
_NS = 16
_NW = _NC * _NS
_NPT = 320
_NP = _NW * _NPT

_FB = 512
_K = 256
_C = 16000
_P = _C + _FB + 16
_ECAP = _E + _FB

_mesh = plsc.VectorSubcoreMesh(core_axis_name="c", subcore_axis_name="s")


def _wid():
    return lax.axis_index("s") * _NC + lax.axis_index("c")


def _scan_body(src_hbm, dst_hbm, lsrc_hbm, ldst_hbm, cnt_hbm,
               src0, src1, dst0, dst1, stg_s, stg_d, out_v, sems0, sems1,
               semd0, semd1):
    w = _wid()
    lo = w * _NPT
    lbase = w * _ECAP
    slots = ((src0, dst0, sems0, semd0), (src1, dst1, sems1, semd1))

    def init_body(i, _):
        dl = (i * 16 + lax.iota(jnp.int32, 16)) % _NPT
        stg_s[pl.ds(i * 16, 16)] = lo + dl
        stg_d[pl.ds(i * 16, 16)] = dl
        return 0
    lax.fori_loop(0, _P // 16, init_body, 0)

    def start_chunk(c, slot):
        src_v, dst_v, sem_s, sem_d = slots[slot]
        pltpu.async_copy(src_hbm.at[pl.ds(c * _C, _C)], src_v, sem_s)
        pltpu.async_copy(dst_hbm.at[pl.ds(c * _C, _C)], dst_v, sem_d)

    def process_chunk(c, slot, carry):
        off, goff = carry
        src_v, dst_v, sem_s, sem_d = slots[slot]
        pltpu.make_async_copy(src_hbm.at[pl.ds(c * _C, _C)], src_v,
                              sem_s).wait()
        pltpu.make_async_copy(dst_hbm.at[pl.ds(c * _C, _C)], dst_v,
                              sem_d).wait()

        _U = 4

        def grp_body(i, off_vec):
            base = _U * i
            st = []
            for u in range(_U):
                s = src_v[pl.ds((base + u) * 16, 16)]
                t = dst_v[pl.ds((base + u) * 16, 16)]
                st.append((s, t))
            ms = [(t >= lo) & (t < lo + _NPT) for (_, t) in st]
            pcs = [plsc.all_reduce_population_count(m) for m in ms]
            css = [plsc.cumsum(m.astype(jnp.int32)) for m in ms]
            acc = off_vec
            for u in range(_U):
                pos = acc + css[u] - 1
                plsc.store_scatter(stg_s, [pos], st[u][0], mask=ms[u])
                plsc.store_scatter(stg_d, [pos], st[u][1] - lo, mask=ms[u])
                acc = acc + pcs[u]
            return acc
        off_vec = jnp.full((16,), 1, jnp.int32) * off
        off = lax.fori_loop(0, _C // 16 // _U, grp_body, off_vec)[0]

        nblk = off // _FB

        def flush_body(j, _):
            o = pl.multiple_of(lbase + goff + j * _FB, _FB)
            pltpu.sync_copy(stg_s.at[pl.ds(j * _FB, _FB)],
                            lsrc_hbm.at[pl.ds(o, _FB)])
            pltpu.sync_copy(stg_d.at[pl.ds(j * _FB, _FB)],
                            ldst_hbm.at[pl.ds(o, _FB)])
            return 0
        lax.fori_loop(0, nblk, flush_body, 0)

        rem = off - nblk * _FB

        def mv_body(i, _):
            vs = stg_s[pl.ds(nblk * _FB + i * 16, 16)]
            vd = stg_d[pl.ds(nblk * _FB + i * 16, 16)]
            stg_s[pl.ds(i * 16, 16)] = vs
            stg_d[pl.ds(i * 16, 16)] = vd
            return 0
        lax.fori_loop(0, (rem + 15) // 16, mv_body, 0)
        return rem, goff + nblk * _FB

    start_chunk(0, 0)

    def pair_body(p, carry):
        a = 2 * p
        start_chunk(a + 1, 1)
        carry = process_chunk(a, 0, carry)

        @pl.when(p + 1 < _E // _C // 2)
        def _():
            start_chunk(a + 2, 0)
        carry = process_chunk(a + 1, 1, carry)
        return carry

    off, goff = lax.fori_loop(0, _E // _C // 2, pair_body,
                              (jnp.int32(0), jnp.int32(0)))

    o = pl.multiple_of(lbase + goff, _FB)
    pltpu.sync_copy(stg_s.at[pl.ds(0, _FB)], lsrc_hbm.at[pl.ds(o, _FB)])
    pltpu.sync_copy(stg_d.at[pl.ds(0, _FB)], ldst_hbm.at[pl.ds(o, _FB)])

    n = goff + off
    out_v[...] = jnp.full((16,), 1, jnp.int32) * n
    pltpu.sync_copy(out_v, cnt_hbm.at[pl.ds(pl.multiple_of(w * 16, 16), 16)])


_scan = functools.partial(
    pl.kernel,
    out_type=[
        jax.ShapeDtypeStruct((_NW * _ECAP,), jnp.int32),
        jax.ShapeDtypeStruct((_NW * _ECAP,), jnp.int32),
        jax.ShapeDtypeStruct((_NW * 16,), jnp.int32),
    ],
    mesh=_mesh,
    compiler_params=pltpu.CompilerParams(needs_layout_passes=False),
    scratch_types=[
        pltpu.VMEM((_C,), jnp.int32),
        pltpu.VMEM((_C,), jnp.int32),
        pltpu.VMEM((_C,), jnp.int32),
        pltpu.VMEM((_C,), jnp.int32),
        pltpu.VMEM((_P,), jnp.int32),
        pltpu.VMEM((_P,), jnp.int32),
        pltpu.VMEM((16,), jnp.int32),
        pltpu.SemaphoreType.DMA,
        pltpu.SemaphoreType.DMA,
        pltpu.SemaphoreType.DMA,
        pltpu.SemaphoreType.DMA,
    ],
)(_scan_body)


def _segmax_body(hp_hbm, lsrc_hbm, ldst_hbm, cnt_hbm, agg_hbm,
                 agg_v, idx0, idx1, dstl0, dstl1, rows0, rows1, cnt_v,
                 sem0, sem1):
    w = _wid()
    lo = w * _NPT
    lbase = w * _ECAP
    slots = ((idx0, dstl0, rows0, sem0), (idx1, dstl1, rows1, sem1))

    pltpu.sync_copy(hp_hbm.at[pl.ds(pl.multiple_of(lo, _NPT), _NPT)], agg_v)
    pltpu.sync_copy(cnt_hbm.at[pl.ds(pl.multiple_of(w * 16, 16), 16)], cnt_v)
    n = cnt_v[pl.ds(0, 16)][0]
    nwin = (n + _K - 1) // _K

    def start(win, slot):
        idx, dstl, rows, sem = slots[slot]
        o = pl.multiple_of(lbase + win * _K, _K)
        pltpu.sync_copy(lsrc_hbm.at[pl.ds(o, _K)], idx)
        pltpu.sync_copy(ldst_hbm.at[pl.ds(o, _K)], dstl)
        pltpu.async_copy(hp_hbm.at[idx], rows, sem)

    def process(slot):
        idx, dstl, rows, sem = slots[slot]
        pltpu.make_async_copy(hp_hbm.at[idx], rows, sem).wait()

        nj = _D // 16

        def grp_body(g, _):
            tv = dstl[pl.ds(g * 16, 16)]
            for l in range(0, 16, 2):
                t1 = tv[l]
                t2 = tv[l + 1]
                k1 = g * 16 + l
                a1 = [agg_v[t1, pl.ds(j * 16, 16)] for j in range(nj)]
                a2 = [agg_v[t2, pl.ds(j * 16, 16)] for j in range(nj)]
                r1 = [rows[k1, pl.ds(j * 16, 16)] for j in range(nj)]
                r2 = [rows[k1 + 1, pl.ds(j * 16, 16)] for j in range(nj)]
                mx1 = [jnp.maximum(a1[j], r1[j]) for j in range(nj)]
                teq = t1 == t2
                mx2 = [jnp.maximum(jnp.where(teq, mx1[j], a2[j]), r2[j])
                       for j in range(nj)]
                for j in range(nj):
                    agg_v[t1, pl.ds(j * 16, 16)] = mx1[j]
                for j in range(nj):
                    agg_v[t2, pl.ds(j * 16, 16)] = mx2[j]
            return 0
        lax.fori_loop(0, _K // 16, grp_body, 0)

    @pl.when(nwin > 0)
    def _():
        start(0, 0)

    def pair_body(p, _):
        a = 2 * p

        @pl.when(a + 1 < nwin)
        def _():
            start(a + 1, 1)
        process(0)

        @pl.when(a + 2 < nwin)
        def _():
            start(a + 2, 0)

        @pl.when(a + 1 < nwin)
        def _():
            process(1)
        return 0
    lax.fori_loop(0, (nwin + 1) // 2, pair_body, 0)

    pltpu.sync_copy(agg_v, agg_hbm.at[pl.ds(pl.multiple_of(lo, _NPT), _NPT)])


_segmax = functools.partial(
    pl.kernel,
    out_type=[jax.ShapeDtypeStruct((_NP, _D), jnp.float32)],
    mesh=_mesh,
    scratch_types=[
        pltpu.VMEM((_NPT, _D), jnp.float32),
        pltpu.VMEM((_K,), jnp.int32),
        pltpu.VMEM((_K,), jnp.int32),
        pltpu.VMEM((_K,), jnp.int32),
        pltpu.VMEM((_K,), jnp.int32),
        pltpu.VMEM((_K, _D), jnp.float32),
        pltpu.VMEM((_K, _D), jnp.float32),
        pltpu.VMEM((16,), jnp.int32),
        pltpu.SemaphoreType.DMA,
        pltpu.SemaphoreType.DMA,
    ],
)(_segmax_body)


_BLK = 2560


def _dense_body(h_ref, agg_ref, W_ref, b_ref, g_ref, be_ref, out_ref):
    h = h_ref[...]
    z = h + agg_ref[...]
    y = lax.dot_general(
        z, W_ref[...], (((1,), (1,)), ((), ())),
        preferred_element_type=jnp.float32,
        precision=lax.Precision.HIGHEST,
    )
    y = y + b_ref[...]
    mu = jnp.mean(y, axis=-1, keepdims=True)
    var = jnp.mean((y - mu) ** 2, axis=-1, keepdims=True)
    y = (y - mu) * lax.rsqrt(var + 1e-5) * g_ref[...] + be_ref[...]
    out_ref[...] = jnp.where(y > 0, y, jnp.exp(jnp.minimum(y, 0.0)) - 1.0)


def _dense(h, agg, W, b, g, be):
    return pl.pallas_call(
        _dense_body,
        grid=(_NP // _BLK,),
        in_specs=[
            pl.BlockSpec((_BLK, _D), lambda i: (i, 0)),
            pl.BlockSpec((_BLK, _D), lambda i: (i, 0)),
            pl.BlockSpec((_D, _D), lambda i: (0, 0)),
            pl.BlockSpec((1, _D), lambda i: (0, 0)),
            pl.BlockSpec((1, _D), lambda i: (0, 0)),
            pl.BlockSpec((1, _D), lambda i: (0, 0)),
        ],
        out_specs=pl.BlockSpec((_BLK, _D), lambda i: (i, 0)),
        out_shape=jax.ShapeDtypeStruct((_NP, _D), jnp.float32),
    )(h, agg, W, b.reshape(1, _D), g.reshape(1, _D), be.reshape(1, _D))


def kernel(x, edge_index, W0, b0, g0, be0, W1, b1, g1, be1, W2, b2, g2, be2):
    src = edge_index[0]
    dst = edge_index[1]
    lsrc, ldst, cnt = _scan(src, dst)
    hp = jnp.pad(x, ((0, _NP - _N), (0, 0)))
    params = [(W0, b0, g0, be0), (W1, b1, g1, be1), (W2, b2, g2, be2)]
    for (W, b, g, be) in params:
        (aggp,) = _segmax(hp, lsrc, ldst, cnt)
        hp = _dense(hp, aggp, W, b, g, be)
    return hp[:_N]


# speedup vs baseline: 1.0514x; 1.0514x over previous
import functools

import jax
import jax.numpy as jnp
from jax import lax
from jax.experimental import pallas as pl
from jax.experimental.pallas import tpu as pltpu
from jax.experimental.pallas import tpu_sc as plsc

_N = 10000
_D = 128
_E = 320000

_NC = 2
_NS = 16
_NW = _NC * _NS
_NPT = 320
_NP = _NW * _NPT

_FB = 512
_K = 256
_C = 16000
_P = _C + _FB + 16
_ECAP = _E + _FB

_mesh = plsc.VectorSubcoreMesh(core_axis_name="c", subcore_axis_name="s")


def _wid():
    return lax.axis_index("s") * _NC + lax.axis_index("c")


def _scan_body(src_hbm, dst_hbm, lsrc_hbm, ldst_hbm, cnt_hbm,
               src0, src1, dst0, dst1, stg_s, stg_d, out_v, sems0, sems1,
               semd0, semd1):
    w = _wid()
    lo = w * _NPT
    lbase = w * _ECAP
    slots = ((src0, dst0, sems0, semd0), (src1, dst1, sems1, semd1))

    def init_body(i, _):
        dl = (i * 16 + lax.iota(jnp.int32, 16)) % _NPT
        stg_s[pl.ds(i * 16, 16)] = lo + dl
        stg_d[pl.ds(i * 16, 16)] = dl
        return 0
    lax.fori_loop(0, _P // 16, init_body, 0)

    def start_chunk(c, slot):
        src_v, dst_v, sem_s, sem_d = slots[slot]
        pltpu.async_copy(src_hbm.at[pl.ds(c * _C, _C)], src_v, sem_s)
        pltpu.async_copy(dst_hbm.at[pl.ds(c * _C, _C)], dst_v, sem_d)

    def process_chunk(c, slot, carry):
        off, goff = carry
        src_v, dst_v, sem_s, sem_d = slots[slot]
        pltpu.make_async_copy(src_hbm.at[pl.ds(c * _C, _C)], src_v,
                              sem_s).wait()
        pltpu.make_async_copy(dst_hbm.at[pl.ds(c * _C, _C)], dst_v,
                              sem_d).wait()

        _U = 8

        def grp_body(i, off_vec):
            base = _U * i
            st = []
            for u in range(_U):
                s = src_v[pl.ds((base + u) * 16, 16)]
                t = dst_v[pl.ds((base + u) * 16, 16)]
                st.append((s, t))
            ms = [(t >= lo) & (t < lo + _NPT) for (_, t) in st]
            pcs = [plsc.all_reduce_population_count(m) for m in ms]
            css = [plsc.cumsum(m.astype(jnp.int32)) for m in ms]
            acc = off_vec
            for u in range(_U):
                pos = acc + css[u] - 1
                plsc.store_scatter(stg_s, [pos], st[u][0], mask=ms[u])
                plsc.store_scatter(stg_d, [pos], st[u][1] - lo, mask=ms[u])
                acc = acc + pcs[u]
            return acc
        off_vec = jnp.full((16,), 1, jnp.int32) * off
        off = lax.fori_loop(0, _C // 16 // _U, grp_body, off_vec)[0]

        nblk = off // _FB

        def flush_body(j, _):
            o = pl.multiple_of(lbase + goff + j * _FB, _FB)
            pltpu.sync_copy(stg_s.at[pl.ds(j * _FB, _FB)],
                            lsrc_hbm.at[pl.ds(o, _FB)])
            pltpu.sync_copy(stg_d.at[pl.ds(j * _FB, _FB)],
                            ldst_hbm.at[pl.ds(o, _FB)])
            return 0
        lax.fori_loop(0, nblk, flush_body, 0)

        rem = off - nblk * _FB

        def mv_body(i, _):
            vs = stg_s[pl.ds(nblk * _FB + i * 16, 16)]
            vd = stg_d[pl.ds(nblk * _FB + i * 16, 16)]
            stg_s[pl.ds(i * 16, 16)] = vs
            stg_d[pl.ds(i * 16, 16)] = vd
            return 0
        lax.fori_loop(0, (rem + 15) // 16, mv_body, 0)
        return rem, goff + nblk * _FB

    start_chunk(0, 0)

    def pair_body(p, carry):
        a = 2 * p
        start_chunk(a + 1, 1)
        carry = process_chunk(a, 0, carry)

        @pl.when(p + 1 < _E // _C // 2)
        def _():
            start_chunk(a + 2, 0)
        carry = process_chunk(a + 1, 1, carry)
        return carry

    off, goff = lax.fori_loop(0, _E // _C // 2, pair_body,
                              (jnp.int32(0), jnp.int32(0)))

    o = pl.multiple_of(lbase + goff, _FB)
    pltpu.sync_copy(stg_s.at[pl.ds(0, _FB)], lsrc_hbm.at[pl.ds(o, _FB)])
    pltpu.sync_copy(stg_d.at[pl.ds(0, _FB)], ldst_hbm.at[pl.ds(o, _FB)])

    n = goff + off
    out_v[...] = jnp.full((16,), 1, jnp.int32) * n
    pltpu.sync_copy(out_v, cnt_hbm.at[pl.ds(pl.multiple_of(w * 16, 16), 16)])


_scan = functools.partial(
    pl.kernel,
    out_type=[
        jax.ShapeDtypeStruct((_NW * _ECAP,), jnp.int32),
        jax.ShapeDtypeStruct((_NW * _ECAP,), jnp.int32),
        jax.ShapeDtypeStruct((_NW * 16,), jnp.int32),
    ],
    mesh=_mesh,
    compiler_params=pltpu.CompilerParams(needs_layout_passes=False),
    scratch_types=[
        pltpu.VMEM((_C,), jnp.int32),
        pltpu.VMEM((_C,), jnp.int32),
        pltpu.VMEM((_C,), jnp.int32),
        pltpu.VMEM((_C,), jnp.int32),
        pltpu.VMEM((_P,), jnp.int32),
        pltpu.VMEM((_P,), jnp.int32),
        pltpu.VMEM((16,), jnp.int32),
        pltpu.SemaphoreType.DMA,
        pltpu.SemaphoreType.DMA,
        pltpu.SemaphoreType.DMA,
        pltpu.SemaphoreType.DMA,
    ],
)(_scan_body)


def _segmax_body(hp_hbm, lsrc_hbm, ldst_hbm, cnt_hbm, agg_hbm,
                 agg_v, idx0, idx1, dstl0, dstl1, rows0, rows1, cnt_v,
                 sem0, sem1):
    w = _wid()
    lo = w * _NPT
    lbase = w * _ECAP
    slots = ((idx0, dstl0, rows0, sem0), (idx1, dstl1, rows1, sem1))

    pltpu.sync_copy(hp_hbm.at[pl.ds(pl.multiple_of(lo, _NPT), _NPT)], agg_v)
    pltpu.sync_copy(cnt_hbm.at[pl.ds(pl.multiple_of(w * 16, 16), 16)], cnt_v)
    n = cnt_v[pl.ds(0, 16)][0]
    nwin = (n + _K - 1) // _K

    def start(win, slot):
        idx, dstl, rows, sem = slots[slot]
        o = pl.multiple_of(lbase + win * _K, _K)
        pltpu.sync_copy(lsrc_hbm.at[pl.ds(o, _K)], idx)
        pltpu.sync_copy(ldst_hbm.at[pl.ds(o, _K)], dstl)
        pltpu.async_copy(hp_hbm.at[idx], rows, sem)

    def process(slot):
        idx, dstl, rows, sem = slots[slot]
        pltpu.make_async_copy(hp_hbm.at[idx], rows, sem).wait()

        def grp_body(g, _):
            tv = dstl[pl.ds(g * 16, 16)]
            for l in range(16):
                t = tv[l]
                k = g * 16 + l
                a = [agg_v[t, pl.ds(j * 16, 16)] for j in range(_D // 16)]
                r = [rows[k, pl.ds(j * 16, 16)] for j in range(_D // 16)]
                mx = [jnp.maximum(a[j], r[j]) for j in range(_D // 16)]
                for j in range(_D // 16):
                    agg_v[t, pl.ds(j * 16, 16)] = mx[j]
            return 0
        lax.fori_loop(0, _K // 16, grp_body, 0)

    @pl.when(nwin > 0)
    def _():
        start(0, 0)

    def pair_body(p, _):
        a = 2 * p

        @pl.when(a + 1 < nwin)
        def _():
            start(a + 1, 1)
        process(0)

        @pl.when(a + 2 < nwin)
        def _():
            start(a + 2, 0)

        @pl.when(a + 1 < nwin)
        def _():
            process(1)
        return 0
    lax.fori_loop(0, (nwin + 1) // 2, pair_body, 0)

    pltpu.sync_copy(agg_v, agg_hbm.at[pl.ds(pl.multiple_of(lo, _NPT), _NPT)])


_segmax = functools.partial(
    pl.kernel,
    out_type=[jax.ShapeDtypeStruct((_NP, _D), jnp.float32)],
    mesh=_mesh,
    scratch_types=[
        pltpu.VMEM((_NPT, _D), jnp.float32),
        pltpu.VMEM((_K,), jnp.int32),
        pltpu.VMEM((_K,), jnp.int32),
        pltpu.VMEM((_K,), jnp.int32),
        pltpu.VMEM((_K,), jnp.int32),
        pltpu.VMEM((_K, _D), jnp.float32),
        pltpu.VMEM((_K, _D), jnp.float32),
        pltpu.VMEM((16,), jnp.int32),
        pltpu.SemaphoreType.DMA,
        pltpu.SemaphoreType.DMA,
    ],
)(_segmax_body)


_BLK = 2560


def _dense_body(h_ref, agg_ref, W_ref, b_ref, g_ref, be_ref, out_ref):
    h = h_ref[...]
    z = h + agg_ref[...]
    y = lax.dot_general(
        z, W_ref[...], (((1,), (1,)), ((), ())),
        preferred_element_type=jnp.float32,
        precision=lax.Precision.HIGHEST,
    )
    y = y + b_ref[...]
    mu = jnp.mean(y, axis=-1, keepdims=True)
    var = jnp.mean((y - mu) ** 2, axis=-1, keepdims=True)
    y = (y - mu) * lax.rsqrt(var + 1e-5) * g_ref[...] + be_ref[...]
    out_ref[...] = jnp.where(y > 0, y, jnp.exp(jnp.minimum(y, 0.0)) - 1.0)


def _dense(h, agg, W, b, g, be):
    return pl.pallas_call(
        _dense_body,
        grid=(_NP // _BLK,),
        in_specs=[
            pl.BlockSpec((_BLK, _D), lambda i: (i, 0)),
            pl.BlockSpec((_BLK, _D), lambda i: (i, 0)),
            pl.BlockSpec((_D, _D), lambda i: (0, 0)),
            pl.BlockSpec((1, _D), lambda i: (0, 0)),
            pl.BlockSpec((1, _D), lambda i: (0, 0)),
            pl.BlockSpec((1, _D), lambda i: (0, 0)),
        ],
        out_specs=pl.BlockSpec((_BLK, _D), lambda i: (i, 0)),
        out_shape=jax.ShapeDtypeStruct((_NP, _D), jnp.float32),
    )(h, agg, W, b.reshape(1, _D), g.reshape(1, _D), be.reshape(1, _D))


def kernel(x, edge_index, W0, b0, g0, be0, W1, b1, g1, be1, W2, b2, g2, be2):
    src = edge_index[0]
    dst = edge_index[1]
    lsrc, ldst, cnt = _scan(src, dst)
    hp = jnp.pad(x, ((0, _NP - _N), (0, 0)))
    params = [(W0, b0, g0, be0), (W1, b1, g1, be1), (W2, b2, g2, be2)]
    for (W, b, g, be) in params:
        (aggp,) = _segmax(hp, lsrc, ldst, cnt)
        hp = _dense(hp, aggp, W, b, g, be)
    return hp[:_N]
